# baseline (speedup 1.0000x reference)
import jax
import jax.numpy as jnp
from jax.experimental import pallas as pl
from jax.experimental.pallas import tpu as pltpu

_BM = 256


def _cdist_block(a_ref, b_ref, out_ref, bbf_ref, b2_ref):
    @pl.when(pl.program_id(0) == 0)
    def _():
        bf = b_ref[...]
        bbf = bf.astype(jnp.bfloat16)
        bbf_ref[...] = bbf
        ones = jnp.ones((1, b_ref.shape[1]), jnp.bfloat16)
        b2_ref[...] = jax.lax.dot_general(
            ones,
            bbf * bbf,
            (((1,), (1,)), ((), ())),
            preferred_element_type=jnp.float32,
        )

    a = a_ref[...]
    a2 = jnp.sum(a * a, axis=1, keepdims=True)
    a_s = (-2.0 * a).astype(jnp.bfloat16)
    mxu = jax.lax.dot_general(
        a_s,
        bbf_ref[...],
        (((1,), (1,)), ((), ())),
        preferred_element_type=jnp.float32,
    )
    d2 = jnp.maximum((a2 + b2_ref[...]) + mxu, 1e-30)
    out_ref[...] = d2 * jax.lax.rsqrt(d2)


def kernel(source_mesh, target_mesh, state):
    del state
    m, k = source_mesh.shape
    n = target_mesh.shape[0]
    return pl.pallas_call(
        _cdist_block,
        grid=(m // _BM,),
        in_specs=[
            pl.BlockSpec((_BM, k), lambda i: (i, 0)),
            pl.BlockSpec((n, k), lambda i: (0, 0)),
        ],
        out_specs=pl.BlockSpec((_BM, n), lambda i: (i, 0)),
        out_shape=jax.ShapeDtypeStruct((m, n), jnp.float32),
        scratch_shapes=[
            pltpu.VMEM((n, k), jnp.bfloat16),
            pltpu.VMEM((1, n), jnp.float32),
        ],
    )(source_mesh, target_mesh)


# speedup vs baseline: 1.0636x; 1.0636x over previous
import jax
import jax.numpy as jnp
from jax.experimental import pallas as pl
from jax.experimental.pallas import tpu as pltpu

_BM = 1024


def _cdist_block(a_ref, b_ref, out_ref, bbf_ref, b2_ref):
    @pl.when(pl.program_id(0) == 0)
    def _():
        bf = b_ref[...]
        bbf = bf.astype(jnp.bfloat16)
        bbf_ref[...] = bbf
        ones = jnp.ones((1, b_ref.shape[1]), jnp.bfloat16)
        b2_ref[...] = jax.lax.dot_general(
            ones,
            bbf * bbf,
            (((1,), (1,)), ((), ())),
            preferred_element_type=jnp.float32,
        )

    a = a_ref[...]
    a2 = jnp.sum(a * a, axis=1, keepdims=True)
    a_s = (-2.0 * a).astype(jnp.bfloat16)
    mxu = jax.lax.dot_general(
        a_s,
        bbf_ref[...],
        (((1,), (1,)), ((), ())),
        preferred_element_type=jnp.float32,
    )
    d2 = jnp.maximum((a2 + b2_ref[...]) + mxu, 1e-30)
    out_ref[...] = d2 * jax.lax.rsqrt(d2)


def kernel(source_mesh, target_mesh, state):
    del state
    m, k = source_mesh.shape
    n = target_mesh.shape[0]
    return pl.pallas_call(
        _cdist_block,
        grid=(m // _BM,),
        in_specs=[
            pl.BlockSpec((_BM, k), lambda i: (i, 0)),
            pl.BlockSpec((n, k), lambda i: (0, 0)),
        ],
        out_specs=pl.BlockSpec((_BM, n), lambda i: (i, 0)),
        out_shape=jax.ShapeDtypeStruct((m, n), jnp.float32),
        scratch_shapes=[
            pltpu.VMEM((n, k), jnp.bfloat16),
            pltpu.VMEM((1, n), jnp.float32),
        ],
    )(source_mesh, target_mesh)


# speedup vs baseline: 1.1421x; 1.0738x over previous
import jax
import jax.numpy as jnp
from jax.experimental import pallas as pl
from jax.experimental.pallas import tpu as pltpu

_BM = 512
_NCH = 4
_OBUF = 3


def _cdist_manual(
    a_hbm,
    b_hbm,
    out_hbm,
    a_vm,
    b_vm,
    bbf_vm,
    b2_vm,
    out_vm,
    a_sems,
    b_sems,
    o_sems,
):
    m, k = a_vm.shape
    n = b_vm.shape[0]
    nb = m // _BM
    ch = n // _NCH

    a_cps = []
    cp = pltpu.make_async_copy(
        a_hbm.at[pl.ds(0, _BM), :], a_vm.at[pl.ds(0, _BM), :], a_sems.at[0]
    )
    cp.start()
    a_cps.append(cp)
    b_cps = []
    for j in range(_NCH):
        cp = pltpu.make_async_copy(
            b_hbm.at[pl.ds(j * ch, ch), :], b_vm.at[pl.ds(j * ch, ch), :], b_sems.at[j]
        )
        cp.start()
        b_cps.append(cp)
    for i in range(1, nb):
        cp = pltpu.make_async_copy(
            a_hbm.at[pl.ds(i * _BM, _BM), :],
            a_vm.at[pl.ds(i * _BM, _BM), :],
            a_sems.at[i],
        )
        cp.start()
        a_cps.append(cp)

    ones = jnp.ones((1, k), jnp.bfloat16)
    out_cps = [None] * nb

    def band_inputs(i):
        a_cps[i].wait()
        a = a_vm[pl.ds(i * _BM, _BM), :]
        a2 = jnp.sum(a * a, axis=1, keepdims=True)
        a_s = (-2.0 * a).astype(jnp.bfloat16)
        return a2, a_s

    a2, a_s = band_inputs(0)
    for j in range(_NCH):
        b_cps[j].wait()
        sl = pl.ds(j * ch, ch)
        c = b_vm[sl, :].astype(jnp.bfloat16)
        bbf_vm[sl, :] = c
        b2c = jax.lax.dot_general(
            ones, c * c, (((1,), (1,)), ((), ())), preferred_element_type=jnp.float32
        )
        b2_vm[:, sl] = b2c
        mxu = jax.lax.dot_general(
            a_s, c, (((1,), (1,)), ((), ())), preferred_element_type=jnp.float32
        )
        d2 = jnp.maximum((a2 + b2c) + mxu, 1e-30)
        out_vm[0, :, sl] = d2 * jax.lax.rsqrt(d2)
    cp = pltpu.make_async_copy(out_vm.at[0], out_hbm.at[pl.ds(0, _BM), :], o_sems.at[0])
    cp.start()
    out_cps[0] = cp

    for i in range(1, nb):
        buf = i % _OBUF
        if i >= _OBUF:
            out_cps[i - _OBUF].wait()
        a2, a_s = band_inputs(i)
        mxu = jax.lax.dot_general(
            a_s,
            bbf_vm[...],
            (((1,), (1,)), ((), ())),
            preferred_element_type=jnp.float32,
        )
        d2 = jnp.maximum((a2 + b2_vm[...]) + mxu, 1e-30)
        out_vm[buf, :, :] = d2 * jax.lax.rsqrt(d2)
        cp = pltpu.make_async_copy(
            out_vm.at[buf], out_hbm.at[pl.ds(i * _BM, _BM), :], o_sems.at[buf]
        )
        cp.start()
        out_cps[i] = cp

    for i in range(nb - _OBUF, nb):
        out_cps[i].wait()


def kernel(source_mesh, target_mesh, state):
    del state
    m, k = source_mesh.shape
    n = target_mesh.shape[0]
    hbm = pl.BlockSpec(memory_space=pltpu.MemorySpace.HBM)
    return pl.pallas_call(
        _cdist_manual,
        in_specs=[hbm, hbm],
        out_specs=hbm,
        out_shape=jax.ShapeDtypeStruct((m, n), jnp.float32),
        scratch_shapes=[
            pltpu.VMEM((m, k), jnp.float32),
            pltpu.VMEM((n, k), jnp.float32),
            pltpu.VMEM((n, k), jnp.bfloat16),
            pltpu.VMEM((1, n), jnp.float32),
            pltpu.VMEM((_OBUF, _BM, n), jnp.float32),
            pltpu.SemaphoreType.DMA((m // _BM,)),
            pltpu.SemaphoreType.DMA((_NCH,)),
            pltpu.SemaphoreType.DMA((_OBUF,)),
        ],
    )(source_mesh, target_mesh)
